# baseline (speedup 1.0000x reference)
import functools

import jax
import jax.numpy as jnp
from jax import lax
from jax.experimental import pallas as pl
from jax.experimental.pallas import tpu as pltpu
from jax.experimental.pallas import tpu_sc as plsc


_COL_START = 26
_COL_COUNT = 100
_N_ROWS = 16384
_NC = 2
_NS = 16
_NW = _NC * _NS
_ROWS_PER_W = _N_ROWS // _NW


def _sc_body(in_hbm, out_hbm, buf_in, buf_out, sem):
    wid = lax.axis_index("s") * _NC + lax.axis_index("c")
    base = wid * _ROWS_PER_W
    rows = pl.ds(base, _ROWS_PER_W)
    pltpu.sync_copy(in_hbm.at[rows, :], buf_in)

    @plsc.parallel_loop(0, _ROWS_PER_W, step=1, unroll=8)
    def row(r):
        for k in range(6):
            buf_out[r, pl.ds(16 * k, 16)] = buf_in[r, pl.ds(_COL_START + 16 * k, 16)]
        buf_out[r, pl.ds(84, 16)] = buf_in[r, pl.ds(_COL_START + 84, 16)]
    pltpu.sync_copy(buf_out, out_hbm.at[rows, :])


_sc_kernel = functools.partial(
    pl.kernel,
    out_type=jax.ShapeDtypeStruct((_N_ROWS, _COL_COUNT), jnp.float32),
    mesh=plsc.VectorSubcoreMesh(core_axis_name="c", subcore_axis_name="s"),
    scratch_types=[
        pltpu.VMEM((_ROWS_PER_W, 126), jnp.float32),
        pltpu.VMEM((_ROWS_PER_W, _COL_COUNT), jnp.float32),
        pltpu.SemaphoreType.DMA,
    ],
)(_sc_body)


def kernel(inputs):
    return _sc_kernel(inputs)


# speedup vs baseline: 2.2920x; 2.2920x over previous
import jax
import jax.numpy as jnp
from jax.experimental import pallas as pl


_COL_START = 26
_COL_COUNT = 100


def _body(in_ref, out_ref):
    out_ref[...] = in_ref[:, _COL_START:_COL_START + _COL_COUNT]


def kernel(inputs):
    n_rows, n_cols = inputs.shape
    block = 8192
    return pl.pallas_call(
        _body,
        grid=(n_rows // block,),
        in_specs=[pl.BlockSpec((block, n_cols), lambda i: (i, 0))],
        out_specs=pl.BlockSpec((block, _COL_COUNT), lambda i: (i, 0)),
        out_shape=jax.ShapeDtypeStruct((n_rows, _COL_COUNT), jnp.float32),
    )(inputs)
